# Initial kernel scaffold; baseline (speedup 1.0000x reference)
import jax
import jax.numpy as jnp
from jax.experimental import pallas as pl


def kernel(sp_dist, table):
    raise NotImplementedError("write your pallas kernel here")


# speedup vs baseline: 10.2520x; 10.2520x over previous
import functools

import jax
import jax.numpy as jnp
from jax import lax
from jax.experimental import pallas as pl
from jax.experimental.pallas import tpu as pltpu
from jax.experimental.pallas import tpu_sc as plsc

B, N, H, K = 4, 512, 32, 22
NC, NS, L = 2, 16, 16
NW = NC * NS
PAIRS = B * N
PPW = PAIRS // NW
C = 4
STEPS = PPW // C


def _sc_body(sp_hbm, tbl_hbm, out_hbm, tbl_v, dbuf, obuf):
    wid = lax.axis_index("s") * NC + lax.axis_index("c")
    pltpu.sync_copy(tbl_hbm, tbl_v)
    pair0 = wid * PPW
    b = pair0 // N
    i_base = pair0 - b * N

    def step_body(step, _):
        base_pair = pair0 + step * C
        pltpu.sync_copy(sp_hbm.at[pl.ds(base_pair * N, C * N)], dbuf)

        for ci in range(C):
            def j_body(jc, _, ci=ci):
                d16 = dbuf[pl.ds(ci * N + jc * L, L)]
                d16 = jnp.minimum(jnp.maximum(d16, 0), K - 1)
                for h in range(H):
                    obuf[h, ci, pl.ds(jc * L, L)] = plsc.load_gather(
                        tbl_v, [d16 + h * K])
                return _

            lax.fori_loop(0, N // L, j_body, None)

        i0 = i_base + step * C
        pltpu.sync_copy(obuf, out_hbm.at[b, :, pl.ds(i0, C), :])
        return _

    lax.fori_loop(0, STEPS, step_body, None)


@jax.jit
def kernel(sp_dist, table):
    tflat = jnp.transpose(table).reshape(-1)
    sp_flat = sp_dist.reshape(-1)
    mesh = plsc.VectorSubcoreMesh(core_axis_name="c", subcore_axis_name="s")
    run = pl.kernel(
        _sc_body,
        out_type=jax.ShapeDtypeStruct((B, H, N, N), jnp.float32),
        mesh=mesh,
        scratch_types=[
            pltpu.VMEM((H * K,), jnp.float32),
            pltpu.VMEM((C * N,), jnp.int32),
            pltpu.VMEM((H, C, N), jnp.float32),
        ],
        compiler_params=pltpu.CompilerParams(needs_layout_passes=False),
    )
    return run(sp_flat, tflat)


# speedup vs baseline: 12.4518x; 1.2146x over previous
import jax
import jax.numpy as jnp
from jax import lax
from jax.experimental import pallas as pl
from jax.experimental.pallas import tpu as pltpu
from jax.experimental.pallas import tpu_sc as plsc

B, N, H, K = 4, 512, 32, 22
NC, NS, L = 2, 16, 16
NW = NC * NS
PAIRS = B * N
PPW = PAIRS // NW
C = 2
STEPS = PPW // C


def _sc_body(sp_hbm, tbl_hbm, out_hbm, tbl_v, dbuf, obuf, isems, osems):
    wid = lax.axis_index("s") * NC + lax.axis_index("c")
    pltpu.sync_copy(tbl_hbm, tbl_v)
    pair0 = wid * PPW
    b = pair0 // N
    i_base = pair0 - b * N

    def start_in(step, sl):
        pltpu.async_copy(
            sp_hbm.at[pl.ds((pair0 + step * C) * N, C * N)],
            dbuf.at[sl], isems.at[sl])

    def start_out(step, sl):
        pltpu.async_copy(
            obuf.at[sl],
            out_hbm.at[b, :, pl.ds(i_base + step * C, C), :], osems.at[sl])

    def wait_in(sl):
        pltpu.make_async_copy(sp_hbm.at[pl.ds(0, C * N)], dbuf.at[sl],
                              isems.at[sl]).wait()

    def wait_out(sl):
        pltpu.make_async_copy(obuf.at[sl],
                              out_hbm.at[0, :, pl.ds(0, C), :],
                              osems.at[sl]).wait()

    def compute(sl):
        for ci in range(C):
            def j_body(jc, carry, ci=ci):
                d16 = dbuf[sl, pl.ds(ci * N + jc * L, L)]
                d16 = jnp.minimum(jnp.maximum(d16, 0), K - 1)
                for h in range(H):
                    obuf[sl, h, ci, pl.ds(jc * L, L)] = plsc.load_gather(
                        tbl_v, [d16 + h * K])
                return carry

            lax.fori_loop(0, N // L, j_body, None)

    start_in(0, 0)
    start_in(1, 1)

    def step_pair(g, carry):
        for sl in range(2):
            step = g * 2 + sl
            wait_in(sl)

            @pl.when(g > 0)
            def _():
                wait_out(sl)

            compute(sl)
            start_out(step, sl)

            @pl.when(step + 2 < STEPS)
            def _():
                start_in(step + 2, sl)
        return carry

    lax.fori_loop(0, STEPS // 2, step_pair, None)
    wait_out(0)
    wait_out(1)


@jax.jit
def kernel(sp_dist, table):
    tflat = jnp.transpose(table).reshape(-1)
    sp_flat = sp_dist.reshape(-1)
    mesh = plsc.VectorSubcoreMesh(core_axis_name="c", subcore_axis_name="s")
    run = pl.kernel(
        _sc_body,
        out_type=jax.ShapeDtypeStruct((B, H, N, N), jnp.float32),
        mesh=mesh,
        scratch_types=[
            pltpu.VMEM((H * K,), jnp.float32),
            pltpu.VMEM((2, C * N), jnp.int32),
            pltpu.VMEM((2, H, C, N), jnp.float32),
            pltpu.SemaphoreType.DMA((2,)),
            pltpu.SemaphoreType.DMA((2,)),
        ],
        compiler_params=pltpu.CompilerParams(needs_layout_passes=False),
    )
    return run(sp_flat, tflat)


# speedup vs baseline: 18.1050x; 1.4540x over previous
import jax
import jax.numpy as jnp
from jax import lax
from jax.experimental import pallas as pl
from jax.experimental.pallas import tpu as pltpu
from jax.experimental.pallas import tpu_sc as plsc

B, N, H, K = 4, 512, 32, 22
NC, NS, L = 2, 16, 16
NW = NC * NS
PAIRS = B * N
PPW = PAIRS // NW
C = 2
CN = C * N
STEPS = PPW // C


def _sc_body(sp_hbm, tbl_hbm, out_hbm, tbl_v, dbuf, obuf, isems, osems):
    wid = lax.axis_index("s") * NC + lax.axis_index("c")
    pltpu.sync_copy(tbl_hbm, tbl_v)
    pair0 = wid * PPW
    b = pair0 // N
    i_base = pair0 - b * N

    def start_in(step, sl):
        pltpu.async_copy(
            sp_hbm.at[pl.ds((pair0 + step * C) * N, CN)],
            dbuf.at[sl], isems.at[sl])

    def start_out(step, sl):
        pltpu.async_copy(
            obuf.at[sl],
            out_hbm.at[b, :, pl.ds((i_base + step * C) * N, CN)],
            osems.at[sl])

    def wait_in(sl):
        pltpu.make_async_copy(sp_hbm.at[pl.ds(0, CN)], dbuf.at[sl],
                              isems.at[sl]).wait()

    def wait_out(sl):
        pltpu.make_async_copy(obuf.at[sl],
                              out_hbm.at[0, :, pl.ds(0, CN)],
                              osems.at[sl]).wait()

    def compute(sl):
        @plsc.parallel_loop(0, CN // L, unroll=2)
        def _(t):
            d16 = dbuf[sl, pl.ds(t * L, L)]
            d16 = jnp.minimum(jnp.maximum(d16, 0), K - 1)
            for h in range(H):
                obuf[sl, h, pl.ds(t * L, L)] = plsc.load_gather(
                    tbl_v, [d16 + h * K])

    start_in(0, 0)
    start_in(1, 1)

    def step_pair(g, carry):
        for sl in range(2):
            step = g * 2 + sl
            wait_in(sl)

            @pl.when(g > 0)
            def _():
                wait_out(sl)

            compute(sl)
            start_out(step, sl)

            @pl.when(step + 2 < STEPS)
            def _():
                start_in(step + 2, sl)
        return carry

    lax.fori_loop(0, STEPS // 2, step_pair, None)
    wait_out(0)
    wait_out(1)


@jax.jit
def kernel(sp_dist, table):
    tflat = jnp.transpose(table).reshape(-1)
    sp_flat = sp_dist.reshape(-1)
    mesh = plsc.VectorSubcoreMesh(core_axis_name="c", subcore_axis_name="s")
    run = pl.kernel(
        _sc_body,
        out_type=jax.ShapeDtypeStruct((B, H, N * N), jnp.float32),
        mesh=mesh,
        scratch_types=[
            pltpu.VMEM((H * K,), jnp.float32),
            pltpu.VMEM((2, CN), jnp.int32),
            pltpu.VMEM((2, H, CN), jnp.float32),
            pltpu.SemaphoreType.DMA((2,)),
            pltpu.SemaphoreType.DMA((2,)),
        ],
        compiler_params=pltpu.CompilerParams(needs_layout_passes=False),
    )
    return run(sp_flat, tflat).reshape(B, H, N, N)


# speedup vs baseline: 20.1041x; 1.1104x over previous
import jax
import jax.numpy as jnp
from jax import lax
from jax.experimental import pallas as pl
from jax.experimental.pallas import tpu as pltpu
from jax.experimental.pallas import tpu_sc as plsc

B, N, H, K = 4, 512, 32, 22
NC, NS, L = 2, 16, 16
NW = NC * NS
PAIRS = B * N
PPW = PAIRS // NW
C = 2
CN = C * N
STEPS = PPW // C


def _sc_body(sp_hbm, tbl_hbm, out_hbm, tbl_v, dbuf, obuf, isems, osems):
    wid = lax.axis_index("s") * NC + lax.axis_index("c")
    pltpu.sync_copy(tbl_hbm, tbl_v)
    pair0 = wid * PPW
    b = pair0 // N
    i_base = pair0 - b * N

    def start_in(step, sl):
        pltpu.async_copy(
            sp_hbm.at[pl.ds((pair0 + step * C) * N, CN)],
            dbuf.at[sl], isems.at[sl])

    def start_out(step, sl):
        pltpu.async_copy(
            obuf.at[sl],
            out_hbm.at[b, :, pl.ds((i_base + step * C) * N, CN)],
            osems.at[sl])

    def wait_in(sl):
        pltpu.make_async_copy(sp_hbm.at[pl.ds(0, CN)], dbuf.at[sl],
                              isems.at[sl]).wait()

    def wait_out(sl):
        pltpu.make_async_copy(obuf.at[sl],
                              out_hbm.at[0, :, pl.ds(0, CN)],
                              osems.at[sl]).wait()

    lane = jnp.arange(L, dtype=jnp.int32)

    def compute(sl):
        @plsc.parallel_loop(0, CN // L, unroll=2)
        def _(t):
            d16 = dbuf[sl, pl.ds(t * L, L)]
            d16 = jnp.minimum(jnp.maximum(d16, 0), K - 1)
            d16s = d16 * L + lane
            for h in range(H):
                obuf[sl, h, pl.ds(t * L, L)] = plsc.load_gather(
                    tbl_v, [d16s + h * (K * L)])

    start_in(0, 0)
    start_in(1, 1)

    def step_pair(g, carry):
        for sl in range(2):
            step = g * 2 + sl
            wait_in(sl)

            @pl.when(g > 0)
            def _():
                wait_out(sl)

            compute(sl)
            start_out(step, sl)

            @pl.when(step + 2 < STEPS)
            def _():
                start_in(step + 2, sl)
        return carry

    lax.fori_loop(0, STEPS // 2, step_pair, None)
    wait_out(0)
    wait_out(1)


@jax.jit
def kernel(sp_dist, table):
    tflat = jnp.transpose(table).reshape(-1)
    trep = jnp.repeat(tflat, L)
    sp_flat = sp_dist.reshape(-1)
    mesh = plsc.VectorSubcoreMesh(core_axis_name="c", subcore_axis_name="s")
    run = pl.kernel(
        _sc_body,
        out_type=jax.ShapeDtypeStruct((B, H, N * N), jnp.float32),
        mesh=mesh,
        scratch_types=[
            pltpu.VMEM((H * K * L,), jnp.float32),
            pltpu.VMEM((2, CN), jnp.int32),
            pltpu.VMEM((2, H, CN), jnp.float32),
            pltpu.SemaphoreType.DMA((2,)),
            pltpu.SemaphoreType.DMA((2,)),
        ],
        compiler_params=pltpu.CompilerParams(needs_layout_passes=False),
    )
    return run(sp_flat, trep).reshape(B, H, N, N)
